# Initial kernel scaffold; baseline (speedup 1.0000x reference)
import functools

import jax
import jax.numpy as jnp
from jax import lax
from jax.experimental import pallas as pl
from jax.experimental.pallas import tpu as pltpu
from jax.experimental.pallas import tpu_sc as plsc

N = 100000
E = 3200000
F = 16
NTILES = 16
EDGES_PER_TILE = E // NTILES
B = 80
NCHUNK = EDGES_PER_TILE // B
ROWS_PER_TILE = N // NTILES
ZROWS = 250
EPS = 1e-16

_mesh = plsc.VectorSubcoreMesh(core_axis_name="c", subcore_axis_name="s")


def _zero_1d(ref, n):
  z = jnp.zeros((16,), jnp.float32)
  def body(i, _):
    ref[pl.ds(i * 16, 16)] = z
    return 0
  lax.fori_loop(0, n // 16, body, 0)
  if n % 16:
    ref[pl.ds(n - 16, 16)] = z


@functools.partial(
    pl.kernel,
    out_type=[
        jax.ShapeDtypeStruct((N, F), jnp.float32),
        jax.ShapeDtypeStruct((N,), jnp.float32),
        jax.ShapeDtypeStruct((N, F), jnp.float32),
        jax.ShapeDtypeStruct((N,), jnp.float32),
    ],
    mesh=_mesh,
    scratch_types=[
        pltpu.VMEM((B,), jnp.int32),
        pltpu.VMEM((B,), jnp.int32),
        pltpu.VMEM((B,), jnp.float32),
        pltpu.VMEM((B,), jnp.float32),
        pltpu.VMEM((B,), jnp.float32),
        pltpu.VMEM((B, F), jnp.float32),
        pltpu.VMEM((ZROWS, F), jnp.float32),
        pltpu.VMEM((ROWS_PER_TILE,), jnp.float32),
        pltpu.VMEM_SHARED((N, F), jnp.float32),
        pltpu.VMEM_SHARED((N,), jnp.float32),
    ],
)
def _sc_layer1(src_h, dst_h, aFs_h, aFd_h, hF_h, aRs_h, aRd_h, hR_h,
               uF_o, denF_o, uR_o, denR_o,
               si, di, a_s, a_d, wv, hrow, zu, zd, u_sh, den_sh):
  cid = lax.axis_index("c")
  sid = lax.axis_index("s")
  r0 = sid * ROWS_PER_TILE

  def zb(i, _):
    zu[i, :] = jnp.zeros((F,), jnp.float32)
    return 0
  lax.fori_loop(0, ZROWS, zb, 0)
  _zero_1d(zd, ROWS_PER_TILE)
  for k in range(ROWS_PER_TILE // ZROWS):
    pltpu.sync_copy(zu, u_sh.at[pl.ds(r0 + k * ZROWS, ZROWS)])
  pltpu.sync_copy(zd, den_sh.at[pl.ds(r0, ROWS_PER_TILE)])
  plsc.subcore_barrier()

  e0 = sid * EDGES_PER_TILE

  def process(s_idx, d_idx, aS_t, aD_t, h_t):
    pltpu.sync_copy(aS_t.at[s_idx], a_s)
    pltpu.sync_copy(aD_t.at[d_idx], a_d)
    pltpu.sync_copy(h_t.at[s_idx], hrow)
    for g in range(B // 16):
      v = a_s[pl.ds(g * 16, 16)] + a_d[pl.ds(g * 16, 16)]
      e = jnp.where(v >= 0.0, v, 0.2 * v)
      wv[pl.ds(g * 16, 16)] = jnp.exp(e)
    for r in range(B):
      hrow[r, :] = hrow[r, :] * wv[r]
    pltpu.sync_copy(hrow, u_sh.at[d_idx], add=True)
    pltpu.sync_copy(wv, den_sh.at[d_idx], add=True)

  def chunk(i, _):
    base = e0 + i * B
    pltpu.sync_copy(src_h.at[pl.ds(base, B)], si)
    pltpu.sync_copy(dst_h.at[pl.ds(base, B)], di)

    @pl.when(cid == 0)
    def _():
      process(si, di, aFs_h, aFd_h, hF_h)

    @pl.when(cid == 1)
    def _():
      process(di, si, aRs_h, aRd_h, hR_h)

    return 0

  lax.fori_loop(0, NCHUNK, chunk, 0)
  plsc.subcore_barrier()

  @pl.when(cid == 0)
  def _():
    for k in range(ROWS_PER_TILE // ZROWS):
      pltpu.sync_copy(u_sh.at[pl.ds(r0 + k * ZROWS, ZROWS)], zu)
      pltpu.sync_copy(zu, uF_o.at[pl.ds(r0 + k * ZROWS, ZROWS)])
    pltpu.sync_copy(den_sh.at[pl.ds(r0, ROWS_PER_TILE)], zd)
    pltpu.sync_copy(zd, denF_o.at[pl.ds(r0, ROWS_PER_TILE)])

  @pl.when(cid == 1)
  def _():
    for k in range(ROWS_PER_TILE // ZROWS):
      pltpu.sync_copy(u_sh.at[pl.ds(r0 + k * ZROWS, ZROWS)], zu)
      pltpu.sync_copy(zu, uR_o.at[pl.ds(r0 + k * ZROWS, ZROWS)])
    pltpu.sync_copy(den_sh.at[pl.ds(r0, ROWS_PER_TILE)], zd)
    pltpu.sync_copy(zd, denR_o.at[pl.ds(r0, ROWS_PER_TILE)])


@functools.partial(
    pl.kernel,
    out_type=[
        jax.ShapeDtypeStruct((N,), jnp.float32),
        jax.ShapeDtypeStruct((N,), jnp.float32),
        jax.ShapeDtypeStruct((N,), jnp.float32),
        jax.ShapeDtypeStruct((N,), jnp.float32),
    ],
    mesh=_mesh,
    scratch_types=[
        pltpu.VMEM((B,), jnp.int32),
        pltpu.VMEM((B,), jnp.int32),
        pltpu.VMEM((B,), jnp.float32),
        pltpu.VMEM((B,), jnp.float32),
        pltpu.VMEM((B,), jnp.float32),
        pltpu.VMEM((B,), jnp.float32),
        pltpu.VMEM((B,), jnp.float32),
        pltpu.VMEM((ROWS_PER_TILE,), jnp.float32),
        pltpu.VMEM_SHARED((N,), jnp.float32),
        pltpu.VMEM_SHARED((N,), jnp.float32),
    ],
)
def _sc_layer2(src_h, dst_h, PF_h, QF_h, TF_h, PR_h, QR_h, TR_h,
               u2F_o, d2F_o, u2R_o, d2R_o,
               si, di, ps, qd, ts, wv, mv, zd, u_sh, den_sh):
  cid = lax.axis_index("c")
  sid = lax.axis_index("s")
  r0 = sid * ROWS_PER_TILE

  _zero_1d(zd, ROWS_PER_TILE)
  pltpu.sync_copy(zd, u_sh.at[pl.ds(r0, ROWS_PER_TILE)])
  pltpu.sync_copy(zd, den_sh.at[pl.ds(r0, ROWS_PER_TILE)])
  plsc.subcore_barrier()

  e0 = sid * EDGES_PER_TILE

  def process(s_idx, d_idx, P_t, Q_t, T_t):
    pltpu.sync_copy(P_t.at[s_idx], ps)
    pltpu.sync_copy(Q_t.at[d_idx], qd)
    pltpu.sync_copy(T_t.at[s_idx], ts)
    for g in range(B // 16):
      sl = pl.ds(g * 16, 16)
      v = ps[sl] + qd[sl]
      e = jnp.where(v >= 0.0, v, 0.2 * v)
      w = jnp.exp(e)
      wv[sl] = w
      mv[sl] = w * ts[sl]
    pltpu.sync_copy(mv, u_sh.at[d_idx], add=True)
    pltpu.sync_copy(wv, den_sh.at[d_idx], add=True)

  def chunk(i, _):
    base = e0 + i * B
    pltpu.sync_copy(src_h.at[pl.ds(base, B)], si)
    pltpu.sync_copy(dst_h.at[pl.ds(base, B)], di)

    @pl.when(cid == 0)
    def _():
      process(si, di, PF_h, QF_h, TF_h)

    @pl.when(cid == 1)
    def _():
      process(di, si, PR_h, QR_h, TR_h)

    return 0

  lax.fori_loop(0, NCHUNK, chunk, 0)
  plsc.subcore_barrier()

  @pl.when(cid == 0)
  def _():
    pltpu.sync_copy(u_sh.at[pl.ds(r0, ROWS_PER_TILE)], zd)
    pltpu.sync_copy(zd, u2F_o.at[pl.ds(r0, ROWS_PER_TILE)])
    pltpu.sync_copy(den_sh.at[pl.ds(r0, ROWS_PER_TILE)], zd)
    pltpu.sync_copy(zd, d2F_o.at[pl.ds(r0, ROWS_PER_TILE)])

  @pl.when(cid == 1)
  def _():
    pltpu.sync_copy(u_sh.at[pl.ds(r0, ROWS_PER_TILE)], zd)
    pltpu.sync_copy(zd, u2R_o.at[pl.ds(r0, ROWS_PER_TILE)])
    pltpu.sync_copy(den_sh.at[pl.ds(r0, ROWS_PER_TILE)], zd)
    pltpu.sync_copy(zd, d2R_o.at[pl.ds(r0, ROWS_PER_TILE)])


BLK = 10000


def _tca_body(x_ref, MF_ref, vasF_ref, vadF_ref, MR_ref, vasR_ref, vadR_ref,
              hF_ref, aFs_ref, aFd_ref, hR_ref, aRs_ref, aRd_ref):
  x = x_ref[...]
  hF_ref[...] = jnp.dot(x, MF_ref[...], preferred_element_type=jnp.float32)
  aFs_ref[...] = jnp.dot(x, vasF_ref[...], preferred_element_type=jnp.float32)
  aFd_ref[...] = jnp.dot(x, vadF_ref[...], preferred_element_type=jnp.float32)
  hR_ref[...] = jnp.dot(x, MR_ref[...], preferred_element_type=jnp.float32)
  aRs_ref[...] = jnp.dot(x, vasR_ref[...], preferred_element_type=jnp.float32)
  aRd_ref[...] = jnp.dot(x, vadR_ref[...], preferred_element_type=jnp.float32)


def _tcb_body(uF_ref, dF_ref, uR_ref, dR_ref, b1_ref, w2F_ref, b1r_ref,
              w2R_ref, sc_ref,
              PF_ref, QF_ref, TF_ref, PR_ref, QR_ref, TR_ref):
  sc = sc_ref[...]
  x1F = jnp.maximum(uF_ref[...] / (dF_ref[...] + EPS) + b1_ref[...], 0.0)
  t2F = jnp.dot(x1F, w2F_ref[...], preferred_element_type=jnp.float32)
  TF_ref[...] = t2F
  PF_ref[...] = t2F * sc[0, 0]
  QF_ref[...] = t2F * sc[0, 1]
  x1R = jnp.maximum(uR_ref[...] / (dR_ref[...] + EPS) + b1r_ref[...], 0.0)
  t2R = jnp.dot(x1R, w2R_ref[...], preferred_element_type=jnp.float32)
  TR_ref[...] = t2R
  PR_ref[...] = t2R * sc[0, 2]
  QR_ref[...] = t2R * sc[0, 3]


def _tcc_body(u2F_ref, d2F_ref, u2R_ref, d2R_ref, bb_ref, out_ref):
  bb = bb_ref[...]
  oF = u2F_ref[...] / (d2F_ref[...] + EPS) + bb[0, 0]
  oR = u2R_ref[...] / (d2R_ref[...] + EPS) + bb[0, 1]
  out_ref[...] = (oF + oR) * 0.5


def _row_spec(cols):
  return pl.BlockSpec((BLK, cols), lambda i: (i, 0))


def _full_spec(shape):
  return pl.BlockSpec(shape, lambda i: tuple(0 for _ in shape))


def kernel(x, edge_index, W1, a1_src, a1_dst, b1, W2, a2_src, a2_dst, b2,
           W1r, a1r_src, a1r_dst, b1r, W2r, a2r_src, a2r_dst, b2r):
  src = edge_index[0]
  dst = edge_index[1]

  MF = W1.T
  vasF = (W1.T @ a1_src)[:, None]
  vadF = (W1.T @ a1_dst)[:, None]
  MR = W1r.T
  vasR = (W1r.T @ a1r_src)[:, None]
  vadR = (W1r.T @ a1r_dst)[:, None]
  w2F = W2.T
  w2R = W2r.T
  sc4 = jnp.stack([a2_src[0], a2_dst[0], a2r_src[0], a2r_dst[0]])[None, :]
  bb2 = jnp.stack([b2[0], b2r[0]])[None, :]

  grid = (N // BLK,)
  f32 = jnp.float32

  hF, aFs, aFd, hR, aRs, aRd = pl.pallas_call(
      _tca_body,
      grid=grid,
      in_specs=[_row_spec(3), _full_spec((3, F)), _full_spec((3, 1)),
                _full_spec((3, 1)), _full_spec((3, F)), _full_spec((3, 1)),
                _full_spec((3, 1))],
      out_specs=[_row_spec(F), _row_spec(1), _row_spec(1),
                 _row_spec(F), _row_spec(1), _row_spec(1)],
      out_shape=[jax.ShapeDtypeStruct((N, F), f32),
                 jax.ShapeDtypeStruct((N, 1), f32),
                 jax.ShapeDtypeStruct((N, 1), f32),
                 jax.ShapeDtypeStruct((N, F), f32),
                 jax.ShapeDtypeStruct((N, 1), f32),
                 jax.ShapeDtypeStruct((N, 1), f32)],
  )(x, MF, vasF, vadF, MR, vasR, vadR)

  uF, denF, uR, denR = _sc_layer1(
      src, dst, aFs.reshape(N), aFd.reshape(N), hF,
      aRs.reshape(N), aRd.reshape(N), hR)

  PF, QF, TF, PR, QR, TR = pl.pallas_call(
      _tcb_body,
      grid=grid,
      in_specs=[_row_spec(F), _row_spec(1), _row_spec(F), _row_spec(1),
                _full_spec((1, F)), _full_spec((F, 1)), _full_spec((1, F)),
                _full_spec((F, 1)), _full_spec((1, 4))],
      out_specs=[_row_spec(1)] * 6,
      out_shape=[jax.ShapeDtypeStruct((N, 1), f32)] * 6,
  )(uF, denF.reshape(N, 1), uR, denR.reshape(N, 1),
    b1[None, :], w2F, b1r[None, :], w2R, sc4)

  u2F, d2F, u2R, d2R = _sc_layer2(
      src, dst, PF.reshape(N), QF.reshape(N), TF.reshape(N),
      PR.reshape(N), QR.reshape(N), TR.reshape(N))

  out = pl.pallas_call(
      _tcc_body,
      grid=grid,
      in_specs=[_row_spec(1), _row_spec(1), _row_spec(1), _row_spec(1),
                _full_spec((1, 2))],
      out_specs=_row_spec(1),
      out_shape=jax.ShapeDtypeStruct((N, 1), f32),
  )(u2F.reshape(N, 1), d2F.reshape(N, 1), u2R.reshape(N, 1),
    d2R.reshape(N, 1), bb2)

  return out


# speedup vs baseline: 36.1398x; 36.1398x over previous
import functools

import jax
import jax.numpy as jnp
from jax import lax
from jax.experimental import pallas as pl
from jax.experimental.pallas import tpu as pltpu
from jax.experimental.pallas import tpu_sc as plsc

N = 100000
E = 3200000
F = 16
NTILES = 16
EDGES_PER_TILE = E // NTILES
B = 80
NCHUNK = EDGES_PER_TILE // B
ROWS_STRIDE = N // NTILES
RPT = 6256
ZROWS = 368
EPS = 1e-16

_mesh = plsc.VectorSubcoreMesh(core_axis_name="c", subcore_axis_name="s")


def _zero_1d(ref, n):
  z = jnp.zeros((16,), jnp.float32)
  def body(i, _):
    ref[pl.ds(i * 16, 16)] = z
    return 0
  lax.fori_loop(0, n // 16, body, 0)
  if n % 16:
    ref[pl.ds(n - 16, 16)] = z


@functools.partial(
    pl.kernel,
    out_type=[
        jax.ShapeDtypeStruct((N, F), jnp.float32),
        jax.ShapeDtypeStruct((N,), jnp.float32),
        jax.ShapeDtypeStruct((N, F), jnp.float32),
        jax.ShapeDtypeStruct((N,), jnp.float32),
    ],
    mesh=_mesh,
    compiler_params=pltpu.CompilerParams(use_tc_tiling_on_sc=False),
    scratch_types=[
        pltpu.VMEM((B,), jnp.int32),
        pltpu.VMEM((B,), jnp.int32),
        pltpu.VMEM((B,), jnp.float32),
        pltpu.VMEM((B,), jnp.float32),
        pltpu.VMEM((B,), jnp.float32),
        pltpu.VMEM((B, F), jnp.float32),
        pltpu.VMEM((ZROWS, F), jnp.float32),
        pltpu.VMEM((RPT,), jnp.float32),
        pltpu.VMEM_SHARED((N, F), jnp.float32),
        pltpu.VMEM_SHARED((N,), jnp.float32),
    ],
)
def _sc_layer1(src_h, dst_h, aFs_h, aFd_h, hF_h, aRs_h, aRd_h, hR_h,
               uF_o, denF_o, uR_o, denR_o,
               si, di, a_s, a_d, wv, hrow, zu, zd, u_sh, den_sh):
  cid = lax.axis_index("c")
  sid = lax.axis_index("s")
  r0 = pl.multiple_of((sid * ROWS_STRIDE) & (-8), 8)

  def zb(i, _):
    zu[i, :] = jnp.zeros((F,), jnp.float32)
    return 0
  lax.fori_loop(0, ZROWS, zb, 0)
  _zero_1d(zd, RPT)
  for k in range(RPT // ZROWS):
    pltpu.sync_copy(zu, u_sh.at[pl.ds(r0 + k * ZROWS, ZROWS)])
  pltpu.sync_copy(zd, den_sh.at[pl.ds(r0, RPT)])
  plsc.subcore_barrier()

  e0 = sid * EDGES_PER_TILE

  def process(s_idx, d_idx, aS_t, aD_t, h_t):
    pltpu.sync_copy(aS_t.at[s_idx], a_s)
    pltpu.sync_copy(aD_t.at[d_idx], a_d)
    pltpu.sync_copy(h_t.at[s_idx], hrow)
    for g in range(B // 16):
      v = a_s[pl.ds(g * 16, 16)] + a_d[pl.ds(g * 16, 16)]
      e = jnp.where(v >= 0.0, v, 0.2 * v)
      w = jnp.exp(e)
      wv[pl.ds(g * 16, 16)] = w
      for j in range(16):
        r = g * 16 + j
        hrow[r, :] = hrow[r, :] * w[j]
    pltpu.sync_copy(hrow, u_sh.at[d_idx], add=True)
    pltpu.sync_copy(wv, den_sh.at[d_idx], add=True)

  def chunk(i, _):
    base = e0 + i * B
    pltpu.sync_copy(src_h.at[pl.ds(base, B)], si)
    pltpu.sync_copy(dst_h.at[pl.ds(base, B)], di)

    @pl.when(cid == 0)
    def _():
      process(si, di, aFs_h, aFd_h, hF_h)

    @pl.when(cid == 1)
    def _():
      process(di, si, aRs_h, aRd_h, hR_h)

    return 0

  lax.fori_loop(0, NCHUNK, chunk, 0)
  plsc.subcore_barrier()

  @pl.when(cid == 0)
  def _():
    for k in range(RPT // ZROWS):
      pltpu.sync_copy(u_sh.at[pl.ds(r0 + k * ZROWS, ZROWS)], zu)
      pltpu.sync_copy(zu, uF_o.at[pl.ds(r0 + k * ZROWS, ZROWS)])
    pltpu.sync_copy(den_sh.at[pl.ds(r0, RPT)], zd)
    pltpu.sync_copy(zd, denF_o.at[pl.ds(r0, RPT)])

  @pl.when(cid == 1)
  def _():
    for k in range(RPT // ZROWS):
      pltpu.sync_copy(u_sh.at[pl.ds(r0 + k * ZROWS, ZROWS)], zu)
      pltpu.sync_copy(zu, uR_o.at[pl.ds(r0 + k * ZROWS, ZROWS)])
    pltpu.sync_copy(den_sh.at[pl.ds(r0, RPT)], zd)
    pltpu.sync_copy(zd, denR_o.at[pl.ds(r0, RPT)])


@functools.partial(
    pl.kernel,
    out_type=[
        jax.ShapeDtypeStruct((N,), jnp.float32),
        jax.ShapeDtypeStruct((N,), jnp.float32),
        jax.ShapeDtypeStruct((N,), jnp.float32),
        jax.ShapeDtypeStruct((N,), jnp.float32),
    ],
    mesh=_mesh,
    compiler_params=pltpu.CompilerParams(use_tc_tiling_on_sc=False),
    scratch_types=[
        pltpu.VMEM((B,), jnp.int32),
        pltpu.VMEM((B,), jnp.int32),
        pltpu.VMEM((B,), jnp.float32),
        pltpu.VMEM((B,), jnp.float32),
        pltpu.VMEM((B,), jnp.float32),
        pltpu.VMEM((B,), jnp.float32),
        pltpu.VMEM((B,), jnp.float32),
        pltpu.VMEM((RPT,), jnp.float32),
        pltpu.VMEM_SHARED((N,), jnp.float32),
        pltpu.VMEM_SHARED((N,), jnp.float32),
    ],
)
def _sc_layer2(src_h, dst_h, PF_h, QF_h, TF_h, PR_h, QR_h, TR_h,
               u2F_o, d2F_o, u2R_o, d2R_o,
               si, di, ps, qd, ts, wv, mv, zd, u_sh, den_sh):
  cid = lax.axis_index("c")
  sid = lax.axis_index("s")
  r0 = pl.multiple_of((sid * ROWS_STRIDE) & (-8), 8)

  _zero_1d(zd, RPT)
  pltpu.sync_copy(zd, u_sh.at[pl.ds(r0, RPT)])
  pltpu.sync_copy(zd, den_sh.at[pl.ds(r0, RPT)])
  plsc.subcore_barrier()

  e0 = sid * EDGES_PER_TILE

  def process(s_idx, d_idx, P_t, Q_t, T_t):
    pltpu.sync_copy(P_t.at[s_idx], ps)
    pltpu.sync_copy(Q_t.at[d_idx], qd)
    pltpu.sync_copy(T_t.at[s_idx], ts)
    for g in range(B // 16):
      sl = pl.ds(g * 16, 16)
      v = ps[sl] + qd[sl]
      e = jnp.where(v >= 0.0, v, 0.2 * v)
      w = jnp.exp(e)
      wv[sl] = w
      mv[sl] = w * ts[sl]
    pltpu.sync_copy(mv, u_sh.at[d_idx], add=True)
    pltpu.sync_copy(wv, den_sh.at[d_idx], add=True)

  def chunk(i, _):
    base = e0 + i * B
    pltpu.sync_copy(src_h.at[pl.ds(base, B)], si)
    pltpu.sync_copy(dst_h.at[pl.ds(base, B)], di)

    @pl.when(cid == 0)
    def _():
      process(si, di, PF_h, QF_h, TF_h)

    @pl.when(cid == 1)
    def _():
      process(di, si, PR_h, QR_h, TR_h)

    return 0

  lax.fori_loop(0, NCHUNK, chunk, 0)
  plsc.subcore_barrier()

  @pl.when(cid == 0)
  def _():
    pltpu.sync_copy(u_sh.at[pl.ds(r0, RPT)], zd)
    pltpu.sync_copy(zd, u2F_o.at[pl.ds(r0, RPT)])
    pltpu.sync_copy(den_sh.at[pl.ds(r0, RPT)], zd)
    pltpu.sync_copy(zd, d2F_o.at[pl.ds(r0, RPT)])

  @pl.when(cid == 1)
  def _():
    pltpu.sync_copy(u_sh.at[pl.ds(r0, RPT)], zd)
    pltpu.sync_copy(zd, u2R_o.at[pl.ds(r0, RPT)])
    pltpu.sync_copy(den_sh.at[pl.ds(r0, RPT)], zd)
    pltpu.sync_copy(zd, d2R_o.at[pl.ds(r0, RPT)])


BLK = 2000


def _tca_body(x_ref, MF_ref, vasF_ref, vadF_ref, MR_ref, vasR_ref, vadR_ref,
              hF_ref, aFs_ref, aFd_ref, hR_ref, aRs_ref, aRd_ref):
  x = x_ref[...]
  hF_ref[...] = jnp.dot(x, MF_ref[...], preferred_element_type=jnp.float32)
  aFs_ref[...] = jnp.dot(x, vasF_ref[...], preferred_element_type=jnp.float32)
  aFd_ref[...] = jnp.dot(x, vadF_ref[...], preferred_element_type=jnp.float32)
  hR_ref[...] = jnp.dot(x, MR_ref[...], preferred_element_type=jnp.float32)
  aRs_ref[...] = jnp.dot(x, vasR_ref[...], preferred_element_type=jnp.float32)
  aRd_ref[...] = jnp.dot(x, vadR_ref[...], preferred_element_type=jnp.float32)


def _tcb_body(uF_ref, dF_ref, uR_ref, dR_ref, b1_ref, w2F_ref, b1r_ref,
              w2R_ref, sc_ref,
              PF_ref, QF_ref, TF_ref, PR_ref, QR_ref, TR_ref):
  sc = sc_ref[...]
  x1F = jnp.maximum(uF_ref[...] / (dF_ref[...] + EPS) + b1_ref[...], 0.0)
  t2F = jnp.dot(x1F, w2F_ref[...], preferred_element_type=jnp.float32)
  TF_ref[...] = t2F
  PF_ref[...] = t2F * sc[0, 0]
  QF_ref[...] = t2F * sc[0, 1]
  x1R = jnp.maximum(uR_ref[...] / (dR_ref[...] + EPS) + b1r_ref[...], 0.0)
  t2R = jnp.dot(x1R, w2R_ref[...], preferred_element_type=jnp.float32)
  TR_ref[...] = t2R
  PR_ref[...] = t2R * sc[0, 2]
  QR_ref[...] = t2R * sc[0, 3]


def _tcc_body(u2F_ref, d2F_ref, u2R_ref, d2R_ref, bb_ref, out_ref):
  bb = bb_ref[...]
  oF = u2F_ref[...] / (d2F_ref[...] + EPS) + bb[0, 0]
  oR = u2R_ref[...] / (d2R_ref[...] + EPS) + bb[0, 1]
  out_ref[...] = (oF + oR) * 0.5


def _row_spec(cols):
  return pl.BlockSpec((BLK, cols), lambda i: (i, 0))


def _full_spec(shape):
  return pl.BlockSpec(shape, lambda i: tuple(0 for _ in shape))


def kernel(x, edge_index, W1, a1_src, a1_dst, b1, W2, a2_src, a2_dst, b2,
           W1r, a1r_src, a1r_dst, b1r, W2r, a2r_src, a2r_dst, b2r):
  src = edge_index[0]
  dst = edge_index[1]

  MF = W1.T
  vasF = (W1.T @ a1_src)[:, None]
  vadF = (W1.T @ a1_dst)[:, None]
  MR = W1r.T
  vasR = (W1r.T @ a1r_src)[:, None]
  vadR = (W1r.T @ a1r_dst)[:, None]
  w2F = W2.T
  w2R = W2r.T
  sc4 = jnp.stack([a2_src[0], a2_dst[0], a2r_src[0], a2r_dst[0]])[None, :]
  bb2 = jnp.stack([b2[0], b2r[0]])[None, :]

  grid = (N // BLK,)
  f32 = jnp.float32

  hF, aFs, aFd, hR, aRs, aRd = pl.pallas_call(
      _tca_body,
      grid=grid,
      in_specs=[_row_spec(3), _full_spec((3, F)), _full_spec((3, 1)),
                _full_spec((3, 1)), _full_spec((3, F)), _full_spec((3, 1)),
                _full_spec((3, 1))],
      out_specs=[_row_spec(F), _row_spec(1), _row_spec(1),
                 _row_spec(F), _row_spec(1), _row_spec(1)],
      out_shape=[jax.ShapeDtypeStruct((N, F), f32),
                 jax.ShapeDtypeStruct((N, 1), f32),
                 jax.ShapeDtypeStruct((N, 1), f32),
                 jax.ShapeDtypeStruct((N, F), f32),
                 jax.ShapeDtypeStruct((N, 1), f32),
                 jax.ShapeDtypeStruct((N, 1), f32)],
  )(x, MF, vasF, vadF, MR, vasR, vadR)

  uF, denF, uR, denR = _sc_layer1(
      src, dst, aFs.reshape(N), aFd.reshape(N), hF,
      aRs.reshape(N), aRd.reshape(N), hR)

  PF, QF, TF, PR, QR, TR = pl.pallas_call(
      _tcb_body,
      grid=grid,
      in_specs=[_row_spec(F), _row_spec(1), _row_spec(F), _row_spec(1),
                _full_spec((1, F)), _full_spec((F, 1)), _full_spec((1, F)),
                _full_spec((F, 1)), _full_spec((1, 4))],
      out_specs=[_row_spec(1)] * 6,
      out_shape=[jax.ShapeDtypeStruct((N, 1), f32)] * 6,
  )(uF, denF.reshape(N, 1), uR, denR.reshape(N, 1),
    b1[None, :], w2F, b1r[None, :], w2R, sc4)

  u2F, d2F, u2R, d2R = _sc_layer2(
      src, dst, PF.reshape(N), QF.reshape(N), TF.reshape(N),
      PR.reshape(N), QR.reshape(N), TR.reshape(N))

  out = pl.pallas_call(
      _tcc_body,
      grid=grid,
      in_specs=[_row_spec(1), _row_spec(1), _row_spec(1), _row_spec(1),
                _full_spec((1, 2))],
      out_specs=_row_spec(1),
      out_shape=jax.ShapeDtypeStruct((N, 1), f32),
  )(u2F.reshape(N, 1), d2F.reshape(N, 1), u2R.reshape(N, 1),
    d2R.reshape(N, 1), bb2)

  return out


# speedup vs baseline: 150.5685x; 4.1663x over previous
import functools

import jax
import jax.numpy as jnp
from jax import lax
from jax.experimental import pallas as pl
from jax.experimental.pallas import tpu as pltpu
from jax.experimental.pallas import tpu_sc as plsc

N = 100000
E = 3200000
F = 16
NTILES = 16
N2 = 100096
RPT = N2 // NTILES
ZROWS = 368
SK = 8
NCHUNK = 198
SK2 = 24
NCHUNK2 = 66
RT = NTILES * SK * NCHUNK
EP = RT * 128
EPS = 1e-16

_mesh = plsc.VectorSubcoreMesh(core_axis_name="c", subcore_axis_name="s")


def _zero_1d(ref, n):
  z = jnp.zeros((16,), jnp.float32)
  def body(i, _):
    ref[pl.ds(i * 16, 16)] = z
    return 0
  lax.fori_loop(0, n // 16, body, 0)


@functools.partial(
    pl.kernel,
    out_type=[
        jax.ShapeDtypeStruct((N2, F), jnp.float32),
        jax.ShapeDtypeStruct((N2,), jnp.float32),
        jax.ShapeDtypeStruct((N2, F), jnp.float32),
        jax.ShapeDtypeStruct((N2,), jnp.float32),
    ],
    mesh=_mesh,
    compiler_params=pltpu.CompilerParams(use_tc_tiling_on_sc=False),
    scratch_types=[
        pltpu.VMEM((SK, 128), jnp.int32),
        pltpu.VMEM((SK, 128), jnp.int32),
        pltpu.VMEM((SK, 128), jnp.float32),
        pltpu.VMEM((SK, 128), jnp.float32),
        pltpu.VMEM((SK, 128), jnp.float32),
        pltpu.VMEM((SK, 128, F), jnp.float32),
        pltpu.VMEM_SHARED((N2, F), jnp.float32),
        pltpu.VMEM_SHARED((N2,), jnp.float32),
        pltpu.SemaphoreType.DMA,
        pltpu.SemaphoreType.DMA,
        pltpu.SemaphoreType.DMA,
    ],
)
def _sc_layer1(src2_h, dst2_h, aFs_h, aFd_h, hF_h, aRs_h, aRd_h, hR_h,
               uF_o, denF_o, uR_o, denR_o,
               si2, di2, as2, ad2, wv2, h3, u_sh, den_sh,
               isem, gsem, ssem):
  cid = lax.axis_index("c")
  sid = lax.axis_index("s")
  r0 = pl.multiple_of(sid * RPT, 8)

  def zrow(r, _):
    h3[0, r, :] = jnp.zeros((F,), jnp.float32)
    return 0
  lax.fori_loop(0, 128, zrow, 0)
  for i in range(8):
    wv2[0, pl.ds(i * 16, 16)] = jnp.zeros((16,), jnp.float32)
  zds = []
  for k in range(49):
    off = r0 + (RPT - 128 if k == 48 else 128 * k)
    zds.append(pltpu.async_copy(h3.at[0], u_sh.at[pl.ds(off, 128)], ssem))
    zds.append(pltpu.async_copy(wv2.at[0], den_sh.at[pl.ds(off, 128)], ssem))
  for d in zds:
    d.wait()
  plsc.subcore_barrier()

  rows0 = sid * (SK * NCHUNK)

  def process(s2, d2, aS_t, aD_t, h_t):
    descs = []
    for j in range(SK):
      descs.append(pltpu.async_copy(aS_t.at[s2.at[j]], as2.at[j], gsem))
      descs.append(pltpu.async_copy(aD_t.at[d2.at[j]], ad2.at[j], gsem))
      descs.append(pltpu.async_copy(h_t.at[s2.at[j]], h3.at[j], gsem))
    for d in descs:
      d.wait()

    def crow(j, _):
      for g in range(8):
        sl = pl.ds(g * 16, 16)
        v = as2[j, sl] + ad2[j, sl]
        e = jnp.where(v >= 0.0, v, 0.2 * v)
        w = jnp.exp(e)
        wv2[j, sl] = w
        for l in range(16):
          r = g * 16 + l
          h3[j, r, :] = h3[j, r, :] * w[l]
      return 0
    lax.fori_loop(0, SK, crow, 0)

    sd = []
    for j in range(SK):
      sd.append(pltpu.async_copy(h3.at[j], u_sh.at[d2.at[j]], ssem, add=True))
      sd.append(pltpu.async_copy(wv2.at[j], den_sh.at[d2.at[j]], ssem,
                                 add=True))
    for d in sd:
      d.wait()

  def chunk(c, _):
    rowbase = rows0 + c * SK
    d1 = pltpu.async_copy(src2_h.at[pl.ds(rowbase, SK)], si2, isem)
    d2_ = pltpu.async_copy(dst2_h.at[pl.ds(rowbase, SK)], di2, isem)
    d1.wait()
    d2_.wait()

    @pl.when(cid == 0)
    def _():
      process(si2, di2, aFs_h, aFd_h, hF_h)

    @pl.when(cid == 1)
    def _():
      process(di2, si2, aRs_h, aRd_h, hR_h)

    return 0

  lax.fori_loop(0, NCHUNK, chunk, 0)
  plsc.subcore_barrier()

  def flush(u_o, den_o):
    for b in range(7):
      js = range(8) if b < 6 else range(1)
      offs = [r0 + (RPT - 128 if b * 8 + j == 48 else 128 * (b * 8 + j))
              for j in js]
      ds_ = []
      for j, off in zip(js, offs):
        ds_.append(pltpu.async_copy(u_sh.at[pl.ds(off, 128)], h3.at[j], gsem))
        ds_.append(pltpu.async_copy(den_sh.at[pl.ds(off, 128)], wv2.at[j],
                                    gsem))
      for d in ds_:
        d.wait()
      ds_ = []
      for j, off in zip(js, offs):
        ds_.append(pltpu.async_copy(h3.at[j], u_o.at[pl.ds(off, 128)], ssem))
        ds_.append(pltpu.async_copy(wv2.at[j], den_o.at[pl.ds(off, 128)],
                                    ssem))
      for d in ds_:
        d.wait()

  @pl.when(cid == 0)
  def _():
    flush(uF_o, denF_o)

  @pl.when(cid == 1)
  def _():
    flush(uR_o, denR_o)


@functools.partial(
    pl.kernel,
    out_type=[
        jax.ShapeDtypeStruct((N2,), jnp.float32),
        jax.ShapeDtypeStruct((N2,), jnp.float32),
        jax.ShapeDtypeStruct((N2,), jnp.float32),
        jax.ShapeDtypeStruct((N2,), jnp.float32),
    ],
    mesh=_mesh,
    compiler_params=pltpu.CompilerParams(use_tc_tiling_on_sc=False),
    scratch_types=[
        pltpu.VMEM((SK2, 128), jnp.int32),
        pltpu.VMEM((SK2, 128), jnp.int32),
        pltpu.VMEM((SK2, 128), jnp.float32),
        pltpu.VMEM((SK2, 128), jnp.float32),
        pltpu.VMEM((SK2, 128), jnp.float32),
        pltpu.VMEM((SK2, 128), jnp.float32),
        pltpu.VMEM((SK2, 128), jnp.float32),
        pltpu.VMEM((RPT,), jnp.float32),
        pltpu.VMEM_SHARED((N2,), jnp.float32),
        pltpu.VMEM_SHARED((N2,), jnp.float32),
        pltpu.SemaphoreType.DMA,
        pltpu.SemaphoreType.DMA,
        pltpu.SemaphoreType.DMA,
    ],
)
def _sc_layer2(src2_h, dst2_h, PF_h, QF_h, TF_h, PR_h, QR_h, TR_h,
               u2F_o, d2F_o, u2R_o, d2R_o,
               si2, di2, ps2, qd2, ts2, wv2, mv2, zd, u_sh, den_sh,
               isem, gsem, ssem):
  cid = lax.axis_index("c")
  sid = lax.axis_index("s")
  r0 = pl.multiple_of(sid * RPT, 8)

  _zero_1d(zd, RPT)
  pltpu.sync_copy(zd, u_sh.at[pl.ds(r0, RPT)])
  pltpu.sync_copy(zd, den_sh.at[pl.ds(r0, RPT)])
  plsc.subcore_barrier()

  rows0 = sid * (SK2 * NCHUNK2)

  def process(s2, d2, P_t, Q_t, T_t):
    descs = []
    for j in range(SK2):
      descs.append(pltpu.async_copy(P_t.at[s2.at[j]], ps2.at[j], gsem))
      descs.append(pltpu.async_copy(Q_t.at[d2.at[j]], qd2.at[j], gsem))
      descs.append(pltpu.async_copy(T_t.at[s2.at[j]], ts2.at[j], gsem))
    for d in descs:
      d.wait()

    def crow(j, _):
      for g in range(8):
        sl = pl.ds(g * 16, 16)
        v = ps2[j, sl] + qd2[j, sl]
        e = jnp.where(v >= 0.0, v, 0.2 * v)
        w = jnp.exp(e)
        wv2[j, sl] = w
        mv2[j, sl] = w * ts2[j, sl]
      return 0
    lax.fori_loop(0, SK2, crow, 0)

    sd = []
    for j in range(SK2):
      sd.append(pltpu.async_copy(mv2.at[j], u_sh.at[d2.at[j]], ssem, add=True))
      sd.append(pltpu.async_copy(wv2.at[j], den_sh.at[d2.at[j]], ssem,
                                 add=True))
    for d in sd:
      d.wait()

  def chunk(c, _):
    rowbase = rows0 + c * SK2
    d1 = pltpu.async_copy(src2_h.at[pl.ds(rowbase, SK2)], si2, isem)
    d2_ = pltpu.async_copy(dst2_h.at[pl.ds(rowbase, SK2)], di2, isem)
    d1.wait()
    d2_.wait()

    @pl.when(cid == 0)
    def _():
      process(si2, di2, PF_h, QF_h, TF_h)

    @pl.when(cid == 1)
    def _():
      process(di2, si2, PR_h, QR_h, TR_h)

    return 0

  lax.fori_loop(0, NCHUNK2, chunk, 0)
  plsc.subcore_barrier()

  @pl.when(cid == 0)
  def _():
    pltpu.sync_copy(u_sh.at[pl.ds(r0, RPT)], zd)
    pltpu.sync_copy(zd, u2F_o.at[pl.ds(r0, RPT)])
    pltpu.sync_copy(den_sh.at[pl.ds(r0, RPT)], zd)
    pltpu.sync_copy(zd, d2F_o.at[pl.ds(r0, RPT)])

  @pl.when(cid == 1)
  def _():
    pltpu.sync_copy(u_sh.at[pl.ds(r0, RPT)], zd)
    pltpu.sync_copy(zd, u2R_o.at[pl.ds(r0, RPT)])
    pltpu.sync_copy(den_sh.at[pl.ds(r0, RPT)], zd)
    pltpu.sync_copy(zd, d2R_o.at[pl.ds(r0, RPT)])


BLK = 3128


def _tca_body(x_ref, MF_ref, vasF_ref, vadF_ref, MR_ref, vasR_ref, vadR_ref,
              hF_ref, aFs_ref, aFd_ref, hR_ref, aRs_ref, aRd_ref):
  x = x_ref[...]
  hF_ref[...] = jnp.dot(x, MF_ref[...], preferred_element_type=jnp.float32)
  aFs_ref[...] = jnp.dot(x, vasF_ref[...], preferred_element_type=jnp.float32)
  aFd_ref[...] = jnp.dot(x, vadF_ref[...], preferred_element_type=jnp.float32)
  hR_ref[...] = jnp.dot(x, MR_ref[...], preferred_element_type=jnp.float32)
  aRs_ref[...] = jnp.dot(x, vasR_ref[...], preferred_element_type=jnp.float32)
  aRd_ref[...] = jnp.dot(x, vadR_ref[...], preferred_element_type=jnp.float32)


def _tcb_body(uF_ref, dF_ref, uR_ref, dR_ref, b1_ref, w2F_ref, b1r_ref,
              w2R_ref, sc_ref,
              PF_ref, QF_ref, TF_ref, PR_ref, QR_ref, TR_ref):
  sc = sc_ref[...]
  x1F = jnp.maximum(uF_ref[...] / (dF_ref[...] + EPS) + b1_ref[...], 0.0)
  t2F = jnp.dot(x1F, w2F_ref[...], preferred_element_type=jnp.float32)
  TF_ref[...] = t2F
  PF_ref[...] = t2F * sc[0, 0]
  QF_ref[...] = t2F * sc[0, 1]
  x1R = jnp.maximum(uR_ref[...] / (dR_ref[...] + EPS) + b1r_ref[...], 0.0)
  t2R = jnp.dot(x1R, w2R_ref[...], preferred_element_type=jnp.float32)
  TR_ref[...] = t2R
  PR_ref[...] = t2R * sc[0, 2]
  QR_ref[...] = t2R * sc[0, 3]


def _tcc_body(u2F_ref, d2F_ref, u2R_ref, d2R_ref, bb_ref, out_ref):
  bb = bb_ref[...]
  oF = u2F_ref[...] / (d2F_ref[...] + EPS) + bb[0, 0]
  oR = u2R_ref[...] / (d2R_ref[...] + EPS) + bb[0, 1]
  out_ref[...] = (oF + oR) * 0.5


def _row_spec(cols):
  return pl.BlockSpec((BLK, cols), lambda i: (i, 0))


def _full_spec(shape):
  return pl.BlockSpec(shape, lambda i: tuple(0 for _ in shape))


def kernel(x, edge_index, W1, a1_src, a1_dst, b1, W2, a2_src, a2_dst, b2,
           W1r, a1r_src, a1r_dst, b1r, W2r, a2r_src, a2r_dst, b2r):
  pad_e = jnp.full((EP - E,), N, jnp.int32)
  src2 = jnp.concatenate([edge_index[0], pad_e]).reshape(RT, 128)
  dst2 = jnp.concatenate([edge_index[1], pad_e]).reshape(RT, 128)
  xp = jnp.pad(x, ((0, N2 - N), (0, 0)))

  MF = W1.T
  vasF = (W1.T @ a1_src)[:, None]
  vadF = (W1.T @ a1_dst)[:, None]
  MR = W1r.T
  vasR = (W1r.T @ a1r_src)[:, None]
  vadR = (W1r.T @ a1r_dst)[:, None]
  w2F = W2.T
  w2R = W2r.T
  sc4 = jnp.stack([a2_src[0], a2_dst[0], a2r_src[0], a2r_dst[0]])[None, :]
  bb2 = jnp.stack([b2[0], b2r[0]])[None, :]

  grid = (N2 // BLK,)
  f32 = jnp.float32

  hF, aFs, aFd, hR, aRs, aRd = pl.pallas_call(
      _tca_body,
      grid=grid,
      in_specs=[_row_spec(3), _full_spec((3, F)), _full_spec((3, 1)),
                _full_spec((3, 1)), _full_spec((3, F)), _full_spec((3, 1)),
                _full_spec((3, 1))],
      out_specs=[_row_spec(F), _row_spec(1), _row_spec(1),
                 _row_spec(F), _row_spec(1), _row_spec(1)],
      out_shape=[jax.ShapeDtypeStruct((N2, F), f32),
                 jax.ShapeDtypeStruct((N2, 1), f32),
                 jax.ShapeDtypeStruct((N2, 1), f32),
                 jax.ShapeDtypeStruct((N2, F), f32),
                 jax.ShapeDtypeStruct((N2, 1), f32),
                 jax.ShapeDtypeStruct((N2, 1), f32)],
  )(xp, MF, vasF, vadF, MR, vasR, vadR)

  uF, denF, uR, denR = _sc_layer1(
      src2, dst2, aFs.reshape(N2), aFd.reshape(N2), hF,
      aRs.reshape(N2), aRd.reshape(N2), hR)

  PF, QF, TF, PR, QR, TR = pl.pallas_call(
      _tcb_body,
      grid=grid,
      in_specs=[_row_spec(F), _row_spec(1), _row_spec(F), _row_spec(1),
                _full_spec((1, F)), _full_spec((F, 1)), _full_spec((1, F)),
                _full_spec((F, 1)), _full_spec((1, 4))],
      out_specs=[_row_spec(1)] * 6,
      out_shape=[jax.ShapeDtypeStruct((N2, 1), f32)] * 6,
  )(uF, denF.reshape(N2, 1), uR, denR.reshape(N2, 1),
    b1[None, :], w2F, b1r[None, :], w2R, sc4)

  u2F, d2F, u2R, d2R = _sc_layer2(
      src2, dst2, PF.reshape(N2), QF.reshape(N2), TF.reshape(N2),
      PR.reshape(N2), QR.reshape(N2), TR.reshape(N2))

  out = pl.pallas_call(
      _tcc_body,
      grid=grid,
      in_specs=[_row_spec(1), _row_spec(1), _row_spec(1), _row_spec(1),
                _full_spec((1, 2))],
      out_specs=_row_spec(1),
      out_shape=jax.ShapeDtypeStruct((N2, 1), f32),
  )(u2F.reshape(N2, 1), d2F.reshape(N2, 1), u2R.reshape(N2, 1),
    d2R.reshape(N2, 1), bb2)

  return out[:N]
